# baseline (speedup 1.0000x reference)
import functools

import jax
import jax.numpy as jnp
from jax import lax
from jax.experimental import pallas as pl
from jax.experimental.pallas import tpu as pltpu
from jax.experimental.pallas import tpu_sc as plsc

NUM_CLASSES = 26
BATCH = 16384
NUM_CORES = 2
NUM_SUBCORES = 16
LANES = 16
NUM_WORKERS = NUM_CORES * NUM_SUBCORES
ROWS_PER_W = BATCH // NUM_WORKERS
GROUPS = ROWS_PER_W // LANES
FLAT_PER_W = ROWS_PER_W * NUM_CLASSES
TAB_PAD = 32


@functools.partial(
    pl.kernel,
    out_type=jax.ShapeDtypeStruct((BATCH,), jnp.int32),
    mesh=plsc.VectorSubcoreMesh(core_axis_name="c", subcore_axis_name="s"),
    compiler_params=pltpu.CompilerParams(needs_layout_passes=False),
    scratch_types=[
        pltpu.VMEM((FLAT_PER_W,), jnp.float32),
        pltpu.VMEM((TAB_PAD,), jnp.int32),
        pltpu.VMEM((ROWS_PER_W,), jnp.int32),
    ],
)
def _prob_to_label_sc(in_hbm, tab_hbm, out_hbm, vals_v, tab_v, out_v):
    wid = lax.axis_index("s") * NUM_CORES + lax.axis_index("c")
    base_row = wid * ROWS_PER_W

    pltpu.sync_copy(in_hbm.at[pl.ds(base_row * NUM_CLASSES, FLAT_PER_W)],
                    vals_v)
    pltpu.sync_copy(tab_hbm, tab_v)

    lane = lax.iota(jnp.int32, LANES)
    row_off = lane * NUM_CLASSES

    def body(g, carry):
        row0 = g * LANES
        base = row0 * NUM_CLASSES + row_off
        best_v = plsc.load_gather(vals_v, [base])
        best_i = jnp.zeros((LANES,), jnp.int32)
        for c in range(1, NUM_CLASSES):
            v = plsc.load_gather(vals_v, [base + c])
            upd = v > best_v
            best_v = jnp.where(upd, v, best_v)
            best_i = jnp.where(upd, jnp.full((LANES,), c, jnp.int32), best_i)
        labels = plsc.load_gather(tab_v, [best_i])
        out_v[pl.ds(row0, LANES)] = labels
        return carry

    lax.fori_loop(0, GROUPS, body, 0)

    pltpu.sync_copy(out_v, out_hbm.at[pl.ds(base_row, ROWS_PER_W)])


def kernel(inputs, label_table):
    flat = inputs.reshape(-1)
    tab = jnp.concatenate(
        [label_table, jnp.zeros((TAB_PAD - NUM_CLASSES,), jnp.int32)])
    return _prob_to_label_sc(flat, tab)


# speedup vs baseline: 1.1117x; 1.1117x over previous
import functools

import jax
import jax.numpy as jnp
from jax import lax
from jax.experimental import pallas as pl
from jax.experimental.pallas import tpu as pltpu
from jax.experimental.pallas import tpu_sc as plsc

NUM_CLASSES = 26
BATCH = 16384
NUM_CORES = 2
NUM_SUBCORES = 16
LANES = 16
NUM_WORKERS = NUM_CORES * NUM_SUBCORES
ROWS_PER_W = BATCH // NUM_WORKERS
GROUPS = ROWS_PER_W // LANES


@functools.partial(
    pl.kernel,
    out_type=jax.ShapeDtypeStruct((BATCH,), jnp.int32),
    mesh=plsc.VectorSubcoreMesh(core_axis_name="c", subcore_axis_name="s"),
    compiler_params=pltpu.CompilerParams(needs_layout_passes=False),
    scratch_types=[
        pltpu.VMEM((ROWS_PER_W, NUM_CLASSES), jnp.float32),
        pltpu.VMEM((NUM_CLASSES,), jnp.int32),
        pltpu.VMEM((ROWS_PER_W,), jnp.int32),
    ],
)
def _prob_to_label_sc(in_hbm, tab_hbm, out_hbm, vals_v, tab_v, out_v):
    wid = lax.axis_index("s") * NUM_CORES + lax.axis_index("c")
    base_row = wid * ROWS_PER_W

    pltpu.sync_copy(in_hbm.at[pl.ds(base_row, ROWS_PER_W), :], vals_v)
    pltpu.sync_copy(tab_hbm, tab_v)

    lane = lax.iota(jnp.int32, LANES)

    def body(g, carry):
        row0 = g * LANES
        rows = row0 + lane
        best_v = plsc.load_gather(vals_v, [rows, jnp.zeros((LANES,), jnp.int32)])
        best_i = jnp.zeros((LANES,), jnp.int32)
        for c in range(1, NUM_CLASSES):
            v = plsc.load_gather(vals_v, [rows, jnp.full((LANES,), c, jnp.int32)])
            upd = v > best_v
            best_v = jnp.where(upd, v, best_v)
            best_i = jnp.where(upd, jnp.full((LANES,), c, jnp.int32), best_i)
        labels = plsc.load_gather(tab_v, [best_i])
        out_v[pl.ds(row0, LANES)] = labels
        return carry

    lax.fori_loop(0, GROUPS, body, 0)

    pltpu.sync_copy(out_v, out_hbm.at[pl.ds(base_row, ROWS_PER_W)])


def kernel(inputs, label_table):
    return _prob_to_label_sc(inputs, label_table)
